# Initial kernel scaffold; baseline (speedup 1.0000x reference)
import jax
import jax.numpy as jnp
from jax.experimental import pallas as pl


def kernel(x_op, x_machine, ei_om_src, ei_om_dst, ei_mo_src, ei_mo_dst, pair_machine, pair_op, params):
    raise NotImplementedError("write your pallas kernel here")


# speedup vs baseline: 1.2000x; 1.2000x over previous
import functools

import jax
import jax.numpy as jnp
from jax.experimental import pallas as pl
from jax.experimental.pallas import tpu as pltpu

_BN_EPS = 1e-5


def _pick_bn(n, cap=2048):
    for bn in (2048, 2000, 1600, 1280, 1024, 1000, 800, 640, 512, 500,
               400, 320, 256, 250, 200, 160, 128, 100, 80, 64, 50, 40,
               32, 25, 16, 8):
        if bn <= cap and n % bn == 0 and bn % 8 == 0:
            return bn
    return n


def _row_spec(bn, d):
    return pl.BlockSpec((bn, d), lambda i: (i, 0))


def _full_spec(shape):
    return pl.BlockSpec(shape, lambda i: tuple(0 for _ in shape))


def _linear1(x, msg, w, b, eps, want_stats):
    n, din = x.shape
    h = w.shape[1]
    bn = _pick_bn(n)
    grid = n // bn

    has_msg = msg is not None
    has_b = b is not None
    has_eps = eps is not None

    def body(*refs):
        it = iter(refs)
        x_ref = next(it)
        msg_ref = next(it) if has_msg else None
        w_ref = next(it)
        b_ref = next(it) if has_b else None
        eps_ref = next(it) if has_eps else None
        o_ref = next(it)
        st_ref = next(it) if want_stats else None
        acc_ref = next(it) if want_stats else None

        g = x_ref[...]
        if has_eps:
            g = (1.0 + eps_ref[0]) * g
        if has_msg:
            g = g + msg_ref[...]
        hh = jnp.dot(g, w_ref[...], preferred_element_type=jnp.float32)
        if has_b:
            hh = hh + b_ref[...]
        o_ref[...] = hh
        if want_stats:
            i = pl.program_id(0)

            @pl.when(i == 0)
            def _():
                acc_ref[...] = jnp.zeros_like(acc_ref)

            acc_ref[0:1, :] += jnp.sum(hh, axis=0, keepdims=True)
            acc_ref[1:2, :] += jnp.sum(hh * hh, axis=0, keepdims=True)

            @pl.when(i == grid - 1)
            def _():
                st_ref[...] = acc_ref[...]

    in_specs = [_row_spec(bn, din)]
    args = [x]
    if has_msg:
        in_specs.append(_row_spec(bn, din))
        args.append(msg)
    in_specs.append(_full_spec((din, h)))
    args.append(w)
    if has_b:
        in_specs.append(_full_spec((1, h)))
        args.append(b.reshape(1, h))
    if has_eps:
        in_specs.append(pl.BlockSpec(memory_space=pltpu.SMEM))
        args.append(eps.reshape(1))

    out_shape = [jax.ShapeDtypeStruct((n, h), jnp.float32)]
    out_specs = [_row_spec(bn, h)]
    scratch = []
    if want_stats:
        out_shape.append(jax.ShapeDtypeStruct((2, h), jnp.float32))
        out_specs.append(_full_spec((2, h)))
        scratch.append(pltpu.VMEM((2, h), jnp.float32))

    res = pl.pallas_call(
        body,
        grid=(grid,),
        in_specs=in_specs,
        out_specs=out_specs if len(out_specs) > 1 else out_specs,
        out_shape=out_shape,
        scratch_shapes=scratch,
    )(*args)
    if want_stats:
        return res[0], res[1]
    return res[0], None


def _bn_relu_linear(h1, stats, nrows, g, be, w, b, res=None, want_stats=False):
    n, h = h1.shape
    h2 = w.shape[1]
    bn = _pick_bn(n)
    grid = n // bn
    has_res = res is not None
    inv_n = 1.0 / float(nrows)

    def body(*refs):
        it = iter(refs)
        h_ref = next(it)
        st_ref = next(it)
        g_ref = next(it)
        be_ref = next(it)
        w_ref = next(it)
        b_ref = next(it)
        res_ref = next(it) if has_res else None
        o_ref = next(it)
        sto_ref = next(it) if want_stats else None
        acc_ref = next(it) if want_stats else None

        mu = st_ref[0:1, :] * inv_n
        var = st_ref[1:2, :] * inv_n - mu * mu
        scale = g_ref[...] * jax.lax.rsqrt(var + _BN_EPS)
        shift = be_ref[...] - mu * scale
        a = jnp.maximum(h_ref[...] * scale + shift, 0.0)
        out = jnp.dot(a, w_ref[...], preferred_element_type=jnp.float32)
        out = out + b_ref[...]
        if has_res:
            out = out + res_ref[...]
        o_ref[...] = out
        if want_stats:
            i = pl.program_id(0)

            @pl.when(i == 0)
            def _():
                acc_ref[...] = jnp.zeros_like(acc_ref)

            acc_ref[0:1, :] += jnp.sum(out, axis=0, keepdims=True)
            acc_ref[1:2, :] += jnp.sum(out * out, axis=0, keepdims=True)

            @pl.when(i == grid - 1)
            def _():
                sto_ref[...] = acc_ref[...]

    in_specs = [
        _row_spec(bn, h),
        _full_spec((2, h)),
        _full_spec((1, h)),
        _full_spec((1, h)),
        _full_spec((h, h2)),
        _full_spec((1, h2)),
    ]
    args = [h1, stats, g.reshape(1, h), be.reshape(1, h), w, b.reshape(1, h2)]
    if has_res:
        in_specs.append(_row_spec(bn, h2))
        args.append(res)

    out_shape = [jax.ShapeDtypeStruct((n, h2), jnp.float32)]
    out_specs = [_row_spec(bn, h2)]
    scratch = []
    if want_stats:
        out_shape.append(jax.ShapeDtypeStruct((2, h2), jnp.float32))
        out_specs.append(_full_spec((2, h2)))
        scratch.append(pltpu.VMEM((2, h2), jnp.float32))

    res_out = pl.pallas_call(
        body,
        grid=(grid,),
        in_specs=in_specs,
        out_specs=out_specs,
        out_shape=out_shape,
        scratch_shapes=scratch,
    )(*args)
    if want_stats:
        return res_out[0], res_out[1]
    return res_out[0], None


def _add_stats(a, b):
    n, h = a.shape
    bn = _pick_bn(n)
    grid = n // bn

    def body(a_ref, b_ref, o_ref, st_ref, acc_ref):
        out = a_ref[...] + b_ref[...]
        o_ref[...] = out
        i = pl.program_id(0)

        @pl.when(i == 0)
        def _():
            acc_ref[...] = jnp.zeros_like(acc_ref)

        acc_ref[0:1, :] += jnp.sum(out, axis=0, keepdims=True)
        acc_ref[1:2, :] += jnp.sum(out * out, axis=0, keepdims=True)

        @pl.when(i == grid - 1)
        def _():
            st_ref[...] = acc_ref[...]

    out, st = pl.pallas_call(
        body,
        grid=(grid,),
        in_specs=[_row_spec(bn, h), _row_spec(bn, h)],
        out_specs=[_row_spec(bn, h), _full_spec((2, h))],
        out_shape=[jax.ShapeDtypeStruct((n, h), jnp.float32),
                   jax.ShapeDtypeStruct((2, h), jnp.float32)],
        scratch_shapes=[pltpu.VMEM((2, h), jnp.float32)],
    )(a, b)
    return out, st


def _bn_relu_final(h2, stats, nrows, g, be, w3, b3):
    n, h = h2.shape
    bn = _pick_bn(n)
    grid = n // bn
    inv_n = 1.0 / float(nrows)

    def body(h_ref, st_ref, g_ref, be_ref, w_ref, b_ref, o_ref):
        mu = st_ref[0:1, :] * inv_n
        var = st_ref[1:2, :] * inv_n - mu * mu
        scale = g_ref[...] * jax.lax.rsqrt(var + _BN_EPS)
        shift = be_ref[...] - mu * scale
        a = jnp.maximum(h_ref[...] * scale + shift, 0.0)
        o_ref[...] = jnp.sum(a * w_ref[...], axis=1, keepdims=True) + b_ref[0]

    out = pl.pallas_call(
        body,
        grid=(grid,),
        in_specs=[
            _row_spec(bn, h),
            _full_spec((2, h)),
            _full_spec((1, h)),
            _full_spec((1, h)),
            _full_spec((1, h)),
            pl.BlockSpec(memory_space=pltpu.SMEM),
        ],
        out_specs=_row_spec(bn, 1),
        out_shape=jax.ShapeDtypeStruct((n, 1), jnp.float32),
    )(h2, stats, g.reshape(1, h), be.reshape(1, h), w3.reshape(1, h), b3)
    return out


def _segment_sum(table, src, dst, num_segments):
    return jax.ops.segment_sum(table[src], dst, num_segments=num_segments)


def _gather_rows(table, idx):
    return table[idx]


def kernel(x_op, x_machine, ei_om_src, ei_om_dst, ei_mo_src, ei_mo_dst,
           pair_machine, pair_op, params):
    n_op = x_op.shape[0]
    n_ma = x_machine.shape[0]
    n_pair = pair_op.shape[0]

    x = {'operation': x_op, 'machine': x_machine}
    resid = None
    for l in range(len(params['layers'])):
        lp = params['layers'][l]
        msg_ma = _segment_sum(x['operation'], ei_om_src, ei_om_dst, n_ma)
        msg_op = _segment_sum(x['machine'], ei_mo_src, ei_mo_dst, n_op)
        new = {}
        for t, msg in (('machine', msg_ma), ('operation', msg_op)):
            p = lp[t]
            nrows = x[t].shape[0]
            h1, st = _linear1(x[t], msg, p['W1'], p['b1'], p['eps'],
                              want_stats=True)
            out, _ = _bn_relu_linear(
                h1, st, nrows, p['g1'], p['be1'], p['W2'], p['b2'],
                res=(resid[t] if resid is not None else None))
            new[t] = out
        resid = new
        x = new

    sp = params['score']
    hh = x['machine'].shape[1]
    w1_ma = sp['W1'][:hh]
    w1_op = sp['W1'][hh:]
    y_ma, _ = _linear1(x['machine'], None, w1_ma, sp['b1'], None,
                       want_stats=False)
    y_op, _ = _linear1(x['operation'], None, w1_op, None, None,
                       want_stats=False)
    gm = _gather_rows(y_ma, pair_machine)
    go = _gather_rows(y_op, pair_op)
    h1, st1 = _add_stats(gm, go)
    h2, st2 = _bn_relu_linear(h1, st1, n_pair, sp['g1'], sp['be1'],
                              sp['W2'], sp['b2'], want_stats=True)
    out = _bn_relu_final(h2, st2, n_pair, sp['g2'], sp['be2'],
                         sp['W3'], sp['b3'])
    return out[:, 0]
